# baseline (speedup 1.0000x reference)
import functools

import jax
import jax.numpy as jnp
from jax import lax
from jax.experimental import pallas as pl
from jax.experimental.pallas import tpu as pltpu
from jax.experimental.pallas import tpu_sc as plsc

N = 10000
E = 320000
NIN = 128
NOUT = 128
FV = 27

NC = 2
NS = 16
NW = NC * NS
EPW = E // NW
CH = 104
NCH = 96
REM_E = EPW - NCH * CH
SLAB = 624
REM_BASE = NS * SLAB
REM_ROWS = N - REM_BASE

BN = 2000
NB = N // BN


def _mm_body(x_ref, w_ref, o_ref):
    o_ref[...] = jnp.dot(
        x_ref[...], w_ref[0], preferred_element_type=jnp.float32
    )[None]


def _transform_features(features, weight):
    return pl.pallas_call(
        _mm_body,
        grid=(NB, FV),
        in_specs=[
            pl.BlockSpec((BN, NIN), lambda n, f: (n, 0)),
            pl.BlockSpec((1, NIN, NOUT), lambda n, f: (f, 0, 0)),
        ],
        out_specs=pl.BlockSpec((1, BN, NOUT), lambda n, f: (f, n, 0)),
        out_shape=jax.ShapeDtypeStruct((FV, N, NOUT), jnp.float32),
    )(features.astype(jnp.bfloat16), weight.astype(jnp.bfloat16))


def _sc_gather_scatter(tf, gidx, didx_main, didx_rem):
    mesh = plsc.VectorSubcoreMesh(core_axis_name="c", subcore_axis_name="s")

    @functools.partial(
        pl.kernel,
        out_type=jax.ShapeDtypeStruct((NC, N, NOUT), jnp.float32),
        mesh=mesh,
        scratch_types=[
            pltpu.VMEM((EPW,), jnp.int32),
            pltpu.VMEM((NCH, CH), jnp.int32),
            pltpu.VMEM((1, REM_E), jnp.int32),
            pltpu.VMEM((CH, NOUT), jnp.float32),
            pltpu.VMEM((CH, NOUT), jnp.float32),
            pltpu.VMEM_SHARED((N, NOUT), jnp.float32),
            pltpu.SemaphoreType.DMA,
            pltpu.SemaphoreType.DMA,
            pltpu.SemaphoreType.DMA,
            pltpu.SemaphoreType.DMA,
        ],
    )
    def sc_kernel(tf_hbm, gi_hbm, di_hbm, dr_hbm, out_hbm,
                  gi_v, di_v, dr_v, rows_a, rows_b, acc_sh, gsa, gsb, ssa, ssb):
        cid = lax.axis_index("c")
        sid = lax.axis_index("s")
        wid = cid * NS + sid

        pltpu.sync_copy(gi_hbm.at[wid], gi_v)
        pltpu.sync_copy(di_hbm.at[wid], di_v)
        pltpu.sync_copy(dr_hbm.at[wid], dr_v)

        @pl.loop(0, CH)
        def _zero_rows(r):
            @pl.loop(0, NOUT, step=16)
            def _zero_lane(c):
                rows_a.at[r, pl.ds(c, 16)][...] = jnp.zeros((16,), jnp.float32)

        base = sid * SLAB

        @pl.loop(0, SLAB, step=48)
        def _zero_slab(r):
            pltpu.sync_copy(rows_a.at[pl.ds(0, 48)], acc_sh.at[pl.ds(base + r, 48)])

        @pl.when(sid == NS - 1)
        def _zero_rem():
            pltpu.sync_copy(
                rows_a.at[pl.ds(0, REM_ROWS)], acc_sh.at[pl.ds(REM_BASE, REM_ROWS)]
            )

        plsc.subcore_barrier()

        pltpu.async_copy(tf_hbm.at[gi_v.at[pl.ds(0, CH)]], rows_a, gsa)
        pltpu.async_copy(tf_hbm.at[gi_v.at[pl.ds(CH, CH)]], rows_b, gsb)

        @pl.loop(0, NCH, step=2)
        def _chunk(j):
            pltpu.make_async_copy(
                tf_hbm.at[gi_v.at[pl.ds(0, CH)]], rows_a, gsa).wait()
            pltpu.async_copy(rows_a, acc_sh.at[di_v.at[j]], ssa, add=True)

            pltpu.make_async_copy(
                tf_hbm.at[gi_v.at[pl.ds(0, CH)]], rows_b, gsb).wait()

            @pl.when(j + 2 < NCH)
            def _refill_a():
                pltpu.make_async_copy(rows_a, acc_sh.at[di_v.at[j]], ssa).wait()
                pltpu.async_copy(
                    tf_hbm.at[gi_v.at[pl.ds((j + 2) * CH, CH)]], rows_a, gsa)

            pltpu.async_copy(rows_b, acc_sh.at[di_v.at[j + 1]], ssb, add=True)

            @pl.when(j + 3 < NCH)
            def _refill_b():
                pltpu.make_async_copy(rows_b, acc_sh.at[di_v.at[j]], ssb).wait()
                pltpu.async_copy(
                    tf_hbm.at[gi_v.at[pl.ds((j + 3) * CH, CH)]], rows_b, gsb)

        pltpu.make_async_copy(rows_a, acc_sh.at[di_v.at[0]], ssa).wait()
        pltpu.make_async_copy(rows_b, acc_sh.at[di_v.at[0]], ssb).wait()

        pltpu.sync_copy(
            tf_hbm.at[gi_v.at[pl.ds(NCH * CH, REM_E)]], rows_a.at[pl.ds(0, REM_E)])
        pltpu.sync_copy(
            rows_a.at[pl.ds(0, REM_E)], acc_sh.at[dr_v.at[0]], add=True)

        plsc.subcore_barrier()

        pltpu.sync_copy(
            acc_sh.at[pl.ds(base, SLAB)],
            out_hbm.at[cid, pl.ds(base, SLAB)],
        )

        @pl.when(sid == NS - 1)
        def _drain_rem():
            pltpu.sync_copy(
                acc_sh.at[pl.ds(REM_BASE, REM_ROWS)],
                out_hbm.at[cid, pl.ds(REM_BASE, REM_ROWS)],
            )

    return sc_kernel(tf, gidx, didx_main, didx_rem)


def _combine_body(p_ref, b_ref, o_ref):
    o_ref[...] = p_ref[0] + p_ref[1] + b_ref[...]


def _combine(partials, bias2d):
    return pl.pallas_call(
        _combine_body,
        grid=(NB,),
        in_specs=[
            pl.BlockSpec((NC, BN, NOUT), lambda n: (0, n, 0)),
            pl.BlockSpec((1, NOUT), lambda n: (0, 0)),
        ],
        out_specs=pl.BlockSpec((BN, NOUT), lambda n: (n, 0)),
        out_shape=jax.ShapeDtypeStruct((N, NOUT), jnp.float32),
    )(partials, bias2d)


def kernel(features, weight, bias, edge_index, offset_id):
    src = edge_index[0].astype(jnp.int32)
    dst = edge_index[1].astype(jnp.int32)
    off = offset_id.astype(jnp.int32)
    gidx = (off * N + src).reshape(NW, EPW)
    didx = dst.reshape(NW, EPW)
    didx_main = didx[:, : NCH * CH].reshape(NW, NCH, CH)
    didx_rem = didx[:, NCH * CH :].reshape(NW, 1, REM_E)
    tf = _transform_features(features, weight).reshape(FV * N, NOUT)
    partials = _sc_gather_scatter(tf, gidx, didx_main, didx_rem)
    return _combine(partials, bias.reshape(1, NOUT))


# speedup vs baseline: 1.2217x; 1.2217x over previous
import functools

import jax
import jax.numpy as jnp
from jax import lax
from jax.experimental import pallas as pl
from jax.experimental.pallas import tpu as pltpu
from jax.experimental.pallas import tpu_sc as plsc

N = 10000
E = 320000
NIN = 128
NOUT = 128
FV = 27

NC = 2
NS = 16
NW = NC * NS
EPW = E // NW
CH = 104
NCH = 96
REM_E = EPW - NCH * CH
SLAB = 624
REM_BASE = NS * SLAB
REM_ROWS = N - REM_BASE

BN = 5000
NB = N // BN


def _mm_body(x_ref, w_ref, o_ref):
    o_ref[...] = jnp.dot(
        x_ref[...], w_ref[0], preferred_element_type=jnp.float32
    )[None]


def _transform_features(features, weight):
    return pl.pallas_call(
        _mm_body,
        grid=(NB, FV),
        in_specs=[
            pl.BlockSpec((BN, NIN), lambda n, f: (n, 0)),
            pl.BlockSpec((1, NIN, NOUT), lambda n, f: (f, 0, 0)),
        ],
        out_specs=pl.BlockSpec((1, BN, NOUT), lambda n, f: (f, n, 0)),
        out_shape=jax.ShapeDtypeStruct((FV, N, NOUT), jnp.float32),
    )(features, weight)


def _sc_gather_scatter(tf, gidx, didx_main, didx_rem):
    mesh = plsc.VectorSubcoreMesh(core_axis_name="c", subcore_axis_name="s")

    @functools.partial(
        pl.kernel,
        out_type=jax.ShapeDtypeStruct((NC, N, NOUT), jnp.float32),
        mesh=mesh,
        scratch_types=[
            pltpu.VMEM((EPW,), jnp.int32),
            pltpu.VMEM((NCH, CH), jnp.int32),
            pltpu.VMEM((1, REM_E), jnp.int32),
            pltpu.VMEM((CH, NOUT), jnp.float32),
            pltpu.VMEM((CH, NOUT), jnp.float32),
            pltpu.VMEM_SHARED((N, NOUT), jnp.float32),
            pltpu.SemaphoreType.DMA,
            pltpu.SemaphoreType.DMA,
            pltpu.SemaphoreType.DMA,
            pltpu.SemaphoreType.DMA,
        ],
    )
    def sc_kernel(tf_hbm, gi_hbm, di_hbm, dr_hbm, out_hbm,
                  gi_v, di_v, dr_v, rows_a, rows_b, acc_sh, gsa, gsb, ssa, ssb):
        cid = lax.axis_index("c")
        sid = lax.axis_index("s")
        wid = cid * NS + sid

        pltpu.sync_copy(gi_hbm.at[wid], gi_v)
        pltpu.sync_copy(di_hbm.at[wid], di_v)
        pltpu.sync_copy(dr_hbm.at[wid], dr_v)

        @pl.loop(0, CH)
        def _zero_rows(r):
            @pl.loop(0, NOUT, step=16)
            def _zero_lane(c):
                rows_a.at[r, pl.ds(c, 16)][...] = jnp.zeros((16,), jnp.float32)

        base = sid * SLAB

        @pl.loop(0, SLAB, step=48)
        def _zero_slab(r):
            pltpu.sync_copy(rows_a.at[pl.ds(0, 48)], acc_sh.at[pl.ds(base + r, 48)])

        @pl.when(sid == NS - 1)
        def _zero_rem():
            pltpu.sync_copy(
                rows_a.at[pl.ds(0, REM_ROWS)], acc_sh.at[pl.ds(REM_BASE, REM_ROWS)]
            )

        plsc.subcore_barrier()

        pltpu.async_copy(tf_hbm.at[gi_v.at[pl.ds(0, CH)]], rows_a, gsa)
        pltpu.async_copy(tf_hbm.at[gi_v.at[pl.ds(CH, CH)]], rows_b, gsb)

        @pl.loop(0, NCH, step=2)
        def _chunk(j):
            pltpu.make_async_copy(
                tf_hbm.at[gi_v.at[pl.ds(0, CH)]], rows_a, gsa).wait()
            pltpu.async_copy(rows_a, acc_sh.at[di_v.at[j]], ssa, add=True)

            pltpu.make_async_copy(
                tf_hbm.at[gi_v.at[pl.ds(0, CH)]], rows_b, gsb).wait()

            @pl.when(j + 2 < NCH)
            def _refill_a():
                pltpu.make_async_copy(rows_a, acc_sh.at[di_v.at[j]], ssa).wait()
                pltpu.async_copy(
                    tf_hbm.at[gi_v.at[pl.ds((j + 2) * CH, CH)]], rows_a, gsa)

            pltpu.async_copy(rows_b, acc_sh.at[di_v.at[j + 1]], ssb, add=True)

            @pl.when(j + 3 < NCH)
            def _refill_b():
                pltpu.make_async_copy(rows_b, acc_sh.at[di_v.at[j]], ssb).wait()
                pltpu.async_copy(
                    tf_hbm.at[gi_v.at[pl.ds((j + 3) * CH, CH)]], rows_b, gsb)

        pltpu.make_async_copy(rows_a, acc_sh.at[di_v.at[0]], ssa).wait()
        pltpu.make_async_copy(rows_b, acc_sh.at[di_v.at[0]], ssb).wait()

        pltpu.sync_copy(
            tf_hbm.at[gi_v.at[pl.ds(NCH * CH, REM_E)]], rows_a.at[pl.ds(0, REM_E)])
        pltpu.sync_copy(
            rows_a.at[pl.ds(0, REM_E)], acc_sh.at[dr_v.at[0]], add=True)

        plsc.subcore_barrier()

        pltpu.sync_copy(
            acc_sh.at[pl.ds(base, SLAB)],
            out_hbm.at[cid, pl.ds(base, SLAB)],
        )

        @pl.when(sid == NS - 1)
        def _drain_rem():
            pltpu.sync_copy(
                acc_sh.at[pl.ds(REM_BASE, REM_ROWS)],
                out_hbm.at[cid, pl.ds(REM_BASE, REM_ROWS)],
            )

    return sc_kernel(tf, gidx, didx_main, didx_rem)


def _combine_body(p_ref, b_ref, o_ref):
    o_ref[...] = p_ref[0] + p_ref[1] + b_ref[...]


def _combine(partials, bias2d):
    return pl.pallas_call(
        _combine_body,
        grid=(NB,),
        in_specs=[
            pl.BlockSpec((NC, BN, NOUT), lambda n: (0, n, 0)),
            pl.BlockSpec((1, NOUT), lambda n: (0, 0)),
        ],
        out_specs=pl.BlockSpec((BN, NOUT), lambda n: (n, 0)),
        out_shape=jax.ShapeDtypeStruct((N, NOUT), jnp.float32),
    )(partials, bias2d)


def kernel(features, weight, bias, edge_index, offset_id):
    src = edge_index[0].astype(jnp.int32)
    dst = edge_index[1].astype(jnp.int32)
    off = offset_id.astype(jnp.int32)
    gidx = (off * N + src).reshape(NW, EPW)
    didx = dst.reshape(NW, EPW)
    didx_main = didx[:, : NCH * CH].reshape(NW, NCH, CH)
    didx_rem = didx[:, NCH * CH :].reshape(NW, 1, REM_E)
    tf = _transform_features(features, weight).reshape(FV * N, NOUT)
    partials = _sc_gather_scatter(tf, gidx, didx_main, didx_rem)
    return _combine(partials, bias.reshape(1, NOUT))


# speedup vs baseline: 1.3577x; 1.1114x over previous
import functools

import jax
import jax.numpy as jnp
from jax import lax
from jax.experimental import pallas as pl
from jax.experimental.pallas import tpu as pltpu
from jax.experimental.pallas import tpu_sc as plsc

N = 10000
E = 320000
NIN = 128
NOUT = 128
FV = 27

NC = 2
NS = 16
NW = NC * NS
EPW = E // NW
CH = 104
NCH = 96
REM_E = EPW - NCH * CH
SLAB = 624
REM_BASE = NS * SLAB
REM_ROWS = N - REM_BASE

BN = 10000
NB = N // BN


def _mm_body(x_ref, w_ref, o_ref):
    o_ref[...] = jnp.dot(
        x_ref[...], w_ref[0], preferred_element_type=jnp.float32
    )[None]


def _transform_features(features, weight):
    return pl.pallas_call(
        _mm_body,
        grid=(NB, FV),
        in_specs=[
            pl.BlockSpec((BN, NIN), lambda n, f: (n, 0)),
            pl.BlockSpec((1, NIN, NOUT), lambda n, f: (f, 0, 0)),
        ],
        out_specs=pl.BlockSpec((1, BN, NOUT), lambda n, f: (f, n, 0)),
        out_shape=jax.ShapeDtypeStruct((FV, N, NOUT), jnp.float32),
    )(features, weight)


def _sc_gather_scatter(tf, gidx, didx_main, didx_rem):
    mesh = plsc.VectorSubcoreMesh(core_axis_name="c", subcore_axis_name="s")

    @functools.partial(
        pl.kernel,
        out_type=jax.ShapeDtypeStruct((NC, N, NOUT), jnp.float32),
        mesh=mesh,
        scratch_types=[
            pltpu.VMEM((EPW,), jnp.int32),
            pltpu.VMEM((NCH, CH), jnp.int32),
            pltpu.VMEM((1, REM_E), jnp.int32),
            pltpu.VMEM((CH, NOUT), jnp.float32),
            pltpu.VMEM((CH, NOUT), jnp.float32),
            pltpu.VMEM_SHARED((N, NOUT), jnp.float32),
            pltpu.SemaphoreType.DMA,
            pltpu.SemaphoreType.DMA,
            pltpu.SemaphoreType.DMA,
            pltpu.SemaphoreType.DMA,
        ],
    )
    def sc_kernel(tf_hbm, gi_hbm, di_hbm, dr_hbm, out_hbm,
                  gi_v, di_v, dr_v, rows_a, rows_b, acc_sh, gsa, gsb, ssa, ssb):
        cid = lax.axis_index("c")
        sid = lax.axis_index("s")
        wid = cid * NS + sid

        pltpu.sync_copy(gi_hbm.at[wid], gi_v)
        pltpu.sync_copy(di_hbm.at[wid], di_v)
        pltpu.sync_copy(dr_hbm.at[wid], dr_v)

        @pl.loop(0, CH)
        def _zero_rows(r):
            @pl.loop(0, NOUT, step=16)
            def _zero_lane(c):
                rows_a.at[r, pl.ds(c, 16)][...] = jnp.zeros((16,), jnp.float32)

        base = sid * SLAB

        @pl.loop(0, SLAB, step=48)
        def _zero_slab(r):
            pltpu.sync_copy(rows_a.at[pl.ds(0, 48)], acc_sh.at[pl.ds(base + r, 48)])

        @pl.when(sid == NS - 1)
        def _zero_rem():
            pltpu.sync_copy(
                rows_a.at[pl.ds(0, REM_ROWS)], acc_sh.at[pl.ds(REM_BASE, REM_ROWS)]
            )

        plsc.subcore_barrier()

        pltpu.async_copy(tf_hbm.at[gi_v.at[pl.ds(0, CH)]], rows_a, gsa)
        pltpu.async_copy(tf_hbm.at[gi_v.at[pl.ds(CH, CH)]], rows_b, gsb)

        @pl.loop(0, NCH, step=2)
        def _chunk(j):
            pltpu.make_async_copy(
                tf_hbm.at[gi_v.at[pl.ds(0, CH)]], rows_a, gsa).wait()
            pltpu.async_copy(rows_a, acc_sh.at[di_v.at[j]], ssa, add=True)

            pltpu.make_async_copy(
                tf_hbm.at[gi_v.at[pl.ds(0, CH)]], rows_b, gsb).wait()

            @pl.when(j + 2 < NCH)
            def _refill_a():
                pltpu.make_async_copy(rows_a, acc_sh.at[di_v.at[j]], ssa).wait()
                pltpu.async_copy(
                    tf_hbm.at[gi_v.at[pl.ds((j + 2) * CH, CH)]], rows_a, gsa)

            pltpu.async_copy(rows_b, acc_sh.at[di_v.at[j + 1]], ssb, add=True)

            @pl.when(j + 3 < NCH)
            def _refill_b():
                pltpu.make_async_copy(rows_b, acc_sh.at[di_v.at[j]], ssb).wait()
                pltpu.async_copy(
                    tf_hbm.at[gi_v.at[pl.ds((j + 3) * CH, CH)]], rows_b, gsb)

        pltpu.make_async_copy(rows_a, acc_sh.at[di_v.at[0]], ssa).wait()
        pltpu.make_async_copy(rows_b, acc_sh.at[di_v.at[0]], ssb).wait()

        pltpu.sync_copy(
            tf_hbm.at[gi_v.at[pl.ds(NCH * CH, REM_E)]], rows_a.at[pl.ds(0, REM_E)])
        pltpu.sync_copy(
            rows_a.at[pl.ds(0, REM_E)], acc_sh.at[dr_v.at[0]], add=True)

        plsc.subcore_barrier()

        pltpu.sync_copy(
            acc_sh.at[pl.ds(base, SLAB)],
            out_hbm.at[cid, pl.ds(base, SLAB)],
        )

        @pl.when(sid == NS - 1)
        def _drain_rem():
            pltpu.sync_copy(
                acc_sh.at[pl.ds(REM_BASE, REM_ROWS)],
                out_hbm.at[cid, pl.ds(REM_BASE, REM_ROWS)],
            )

    return sc_kernel(tf, gidx, didx_main, didx_rem)


def _combine_body(p_ref, b_ref, o_ref):
    o_ref[...] = p_ref[0] + p_ref[1] + b_ref[...]


def _combine(partials, bias2d):
    return pl.pallas_call(
        _combine_body,
        grid=(NB,),
        in_specs=[
            pl.BlockSpec((NC, BN, NOUT), lambda n: (0, n, 0)),
            pl.BlockSpec((1, NOUT), lambda n: (0, 0)),
        ],
        out_specs=pl.BlockSpec((BN, NOUT), lambda n: (n, 0)),
        out_shape=jax.ShapeDtypeStruct((N, NOUT), jnp.float32),
    )(partials, bias2d)


def kernel(features, weight, bias, edge_index, offset_id):
    src = edge_index[0].astype(jnp.int32)
    dst = edge_index[1].astype(jnp.int32)
    off = offset_id.astype(jnp.int32)
    gidx = (off * N + src).reshape(NW, EPW)
    didx = dst.reshape(NW, EPW)
    didx_main = didx[:, : NCH * CH].reshape(NW, NCH, CH)
    didx_rem = didx[:, NCH * CH :].reshape(NW, 1, REM_E)
    tf = _transform_features(features, weight).reshape(FV * N, NOUT)
    partials = _sc_gather_scatter(tf, gidx, didx_main, didx_rem)
    return _combine(partials, bias.reshape(1, NOUT))
